# Initial kernel scaffold; baseline (speedup 1.0000x reference)
import jax
import jax.numpy as jnp
from jax.experimental import pallas as pl


def kernel(filtre, X, node_indicator):
    raise NotImplementedError("write your pallas kernel here")


# speedup vs baseline: 4.7983x; 4.7983x over previous
import functools

import jax
import jax.numpy as jnp
from jax.experimental import pallas as pl
from jax.experimental.pallas import tpu as pltpu

NUM_SEG = 1024
N_ROWS = 320000
D = 128
BLK = 1280
NB = N_ROWS // BLK


def _pool_body(ids_ref, x_ref, out_ref, acc_ref, cnt_ref):
    i = pl.program_id(0)

    @pl.when(i == 0)
    def _init():
        acc_ref[...] = jnp.zeros_like(acc_ref)
        cnt_ref[...] = jnp.zeros_like(cnt_ref)

    ids = ids_ref[0, 0, :]
    seg = jax.lax.broadcasted_iota(jnp.int32, (NUM_SEG, BLK), 0)
    onehot = (seg == ids[None, :]).astype(jnp.float32)
    acc_ref[...] += jax.lax.dot(
        onehot, x_ref[...], preferred_element_type=jnp.float32
    )
    cnt_ref[...] += jnp.sum(onehot, axis=1, keepdims=True)

    @pl.when(i == NB - 1)
    def _fin():
        out_ref[...] = acc_ref[...] / jnp.maximum(cnt_ref[...], 1.0)


@jax.jit
def _pool(X, ids):
    ids3 = ids.astype(jnp.int32).reshape(NB, 1, BLK)
    return pl.pallas_call(
        _pool_body,
        grid=(NB,),
        in_specs=[
            pl.BlockSpec((1, 1, BLK), lambda i: (i, 0, 0)),
            pl.BlockSpec((BLK, D), lambda i: (i, 0)),
        ],
        out_specs=pl.BlockSpec((NUM_SEG, D), lambda i: (0, 0)),
        out_shape=jax.ShapeDtypeStruct((NUM_SEG, D), jnp.float32),
        scratch_shapes=[
            pltpu.VMEM((NUM_SEG, D), jnp.float32),
            pltpu.VMEM((NUM_SEG, 1), jnp.float32),
        ],
    )(ids3, X)


def kernel(filtre, X, node_indicator):
    return (filtre, _pool(X, node_indicator))


# speedup vs baseline: 8.6298x; 1.7985x over previous
import functools

import jax
import jax.numpy as jnp
from jax import lax
from jax.experimental import pallas as pl
from jax.experimental.pallas import tpu as pltpu
from jax.experimental.pallas import tpu_sc as plsc

NUM_SEG = 1024
N_ROWS = 320000
D = 128
NC = 2
NS = 16
NW = NC * NS
CHUNK = N_ROWS // NW
BLK = 400
NBLK = CHUNK // BLK
PAD = 16
NVEC = CHUNK // 16
STRIPE = NUM_SEG // NS


def _sc_body(x_hbm, ids_hbm, out_acc, out_cnt,
             ids_v, buf, bpos, flushb, cflush, idx1, zbuf, sacc, scnt):
    cid = lax.axis_index("c")
    sid = lax.axis_index("s")
    w = cid * NS + sid
    z16 = jnp.zeros((16,), jnp.float32)

    def _z(r, carry):
        for j in range(D // 16):
            zbuf[r, pl.ds(j * 16, 16)] = z16
        return carry

    lax.fori_loop(0, STRIPE, _z, 0)
    pltpu.sync_copy(zbuf, sacc.at[pl.ds(sid * STRIPE, STRIPE)])
    pltpu.sync_copy(zbuf, scnt.at[pl.ds(sid * STRIPE, STRIPE)])
    plsc.subcore_barrier()

    pltpu.sync_copy(ids_hbm.at[pl.ds(w * CHUNK, CHUNK)], ids_v.at[pl.ds(PAD, CHUNK)])
    iota16 = lax.iota(jnp.int32, 16)
    lane0 = iota16 == 0
    first = ids_v[pl.ds(PAD, 16)][0]
    plsc.store_scatter(ids_v, [jnp.broadcast_to(jnp.int32(PAD - 1), (16,))],
                       jnp.broadcast_to(first - 1, (16,)), mask=lane0)

    def _scan(i, off):
        base = i * 16
        c = ids_v[pl.ds(base + PAD, 16)]
        p = ids_v[pl.ds(base + PAD - 1, 16)]
        m = c != p
        m32 = m.astype(jnp.int32)
        excl = plsc.cumsum(m32) - m32
        plsc.store_scatter(bpos, [off + excl], base + iota16, mask=m)
        return off + jnp.sum(m32)

    nb = lax.fori_loop(0, NVEC, _scan, jnp.int32(0))
    plsc.store_scatter(bpos, [jnp.broadcast_to(nb, (16,))],
                       jnp.broadcast_to(jnp.int32(CHUNK), (16,)), mask=lane0)

    def _flush(rs, re, acc):
        for j in range(D // 16):
            flushb[0, pl.ds(j * 16, 16)] = acc[j]
        cnt = jnp.broadcast_to((re - rs).astype(jnp.float32), (16,))
        for j in range(D // 16):
            cflush[0, pl.ds(j * 16, 16)] = cnt
        seg = ids_v[pl.ds(rs + PAD, 16)][0]
        plsc.store_scatter(idx1, [jnp.zeros((16,), jnp.int32)],
                           jnp.broadcast_to(seg, (16,)), mask=lane0)
        pltpu.sync_copy(flushb, sacc.at[idx1], add=True)
        pltpu.sync_copy(cflush, scnt.at[idx1], add=True)

    def _block(b, st):
        lo = b * BLK
        hi = lo + BLK
        pltpu.sync_copy(x_hbm.at[pl.ds(w * CHUNK + lo, BLK)], buf)

        def _cond(s):
            return s[1] < hi

        def _piece(s):
            k, pos = s[0], s[1]
            acc = s[2:]
            bv = bpos[pl.ds(k, 16)]
            rs, re = bv[0], bv[1]
            pe = jnp.minimum(re, hi)

            def _row(r, a):
                return tuple(a[j] + buf[r - lo, pl.ds(j * 16, 16)]
                             for j in range(D // 16))

            acc = lax.fori_loop(pos, pe, _row, acc)
            run_done = pe == re

            def _tb(a):
                _flush(rs, re, a)
                return tuple(z16 for _ in range(D // 16))

            acc = lax.cond(run_done, _tb, lambda a: a, acc)
            k = jnp.where(run_done, k + 1, k)
            return (k, pe) + acc

        return lax.while_loop(_cond, _piece, st)

    st0 = (jnp.int32(0), jnp.int32(0)) + tuple(z16 for _ in range(D // 16))
    lax.fori_loop(0, NBLK, _block, st0)
    plsc.subcore_barrier()

    pltpu.sync_copy(sacc.at[pl.ds(sid * STRIPE, STRIPE)], zbuf)
    pltpu.sync_copy(zbuf, out_acc.at[cid, pl.ds(sid * STRIPE, STRIPE)])
    pltpu.sync_copy(scnt.at[pl.ds(sid * STRIPE, STRIPE)], zbuf)
    pltpu.sync_copy(zbuf, out_cnt.at[cid, pl.ds(sid * STRIPE, STRIPE)])


_sc_pool = pl.kernel(
    _sc_body,
    out_type=(
        jax.ShapeDtypeStruct((NC, NUM_SEG, D), jnp.float32),
        jax.ShapeDtypeStruct((NC, NUM_SEG, D), jnp.float32),
    ),
    mesh=plsc.VectorSubcoreMesh(core_axis_name="c", subcore_axis_name="s"),
    compiler_params=pltpu.CompilerParams(needs_layout_passes=False),
    scratch_types=[
        pltpu.VMEM((CHUNK + PAD + 16,), jnp.int32),
        pltpu.VMEM((BLK, D), jnp.float32),
        pltpu.VMEM((NUM_SEG + 48,), jnp.int32),
        pltpu.VMEM((1, D), jnp.float32),
        pltpu.VMEM((1, D), jnp.float32),
        pltpu.VMEM((1,), jnp.int32),
        pltpu.VMEM((STRIPE, D), jnp.float32),
        pltpu.VMEM_SHARED((NUM_SEG, D), jnp.float32),
        pltpu.VMEM_SHARED((NUM_SEG, D), jnp.float32),
    ],
)


def _combine_body(a_ref, c_ref, o_ref):
    a = a_ref[0] + a_ref[1]
    c = c_ref[0] + c_ref[1]
    o_ref[...] = a / jnp.maximum(c, 1.0)


@jax.jit
def _pool(X, ids):
    acc, cnt = _sc_pool(X, ids)
    return pl.pallas_call(
        _combine_body,
        out_shape=jax.ShapeDtypeStruct((NUM_SEG, D), jnp.float32),
    )(acc, cnt)


def kernel(filtre, X, node_indicator):
    return (filtre, _pool(X, node_indicator.astype(jnp.int32)))
